# Initial kernel scaffold; baseline (speedup 1.0000x reference)
import jax
import jax.numpy as jnp
from jax.experimental import pallas as pl


def kernel(x, Wg1, bg1, Wg2, bg2, W1, b1, W2, b2, W3, b3):
    raise NotImplementedError("write your pallas kernel here")


# speedup vs baseline: 1.5912x; 1.5912x over previous
import functools

import jax
import jax.numpy as jnp
import numpy as np
from jax.experimental import pallas as pl

_EPS = 1e-08
_K = 8
_TN = 512


def _mm(a, b, prec):
    return jax.lax.dot_general(
        a, b, (((a.ndim - 1,), (0,)), ((), ())),
        precision=prec, preferred_element_type=jnp.float32)


def _collapse_body(W1_ref, W2_ref, W3_ref, b1_ref, b2_ref, b3_ref,
                   Wcat_ref, bcat_ref):
    hi = jax.lax.Precision.HIGHEST
    eg = W1_ref.shape[0]
    wcols, bcols = [], []
    for i in range(eg):
        W23 = _mm(W2_ref[i], W3_ref[i], hi)
        wcols.append(_mm(W1_ref[i], W23, hi))
        bc = _mm(b1_ref[i], W23, hi) + _mm(b2_ref[i], W3_ref[i], hi) + b3_ref[i]
        bcols.append(bc)
    Wcat_ref[...] = jnp.concatenate(wcols, axis=1)
    bcat_ref[...] = jnp.concatenate(bcols, axis=1)


def _moe_body(x_ref, Wg1_ref, bg1_ref, Wg2_ref, bg2_ref, Wcat_ref, bcat_ref,
              noise_ref, R_ref, S_ref, out_ref):
    hi = jax.lax.Precision.HIGHEST
    df = jax.lax.Precision.DEFAULT
    x = x_ref[...]

    hg = jnp.maximum(_mm(x, Wg1_ref[...], hi) + bg1_ref[...], 0.0)
    logits = _mm(hg, Wg2_ref[...], hi) + bg2_ref[...]
    m = jnp.max(logits, axis=-1, keepdims=True)
    ex = jnp.exp(logits - m)
    w = ex / jnp.sum(ex, axis=-1, keepdims=True)
    w = w + noise_ref[0, 0]

    e_dim = w.shape[-1]
    iota = jax.lax.broadcasted_iota(jnp.int32, w.shape, 1)
    wk = w
    mask = jnp.zeros(w.shape, jnp.bool_)
    for _ in range(_K):
        mx = jnp.max(wk, axis=-1, keepdims=True)
        cand = jnp.where(wk == mx, iota, e_dim)
        sel = jnp.min(cand, axis=-1, keepdims=True)
        first = iota == sel
        mask = jnp.logical_or(mask, first)
        wk = jnp.where(first, -jnp.inf, wk)
    wsel = jnp.where(mask, w, 0.0)

    P = _mm(x, Wcat_ref[...], df)
    wide = _mm(wsel, R_ref[...], hi)
    fin = _mm(P * wide, S_ref[...], df)
    fin = fin + _mm(wsel, bcat_ref[...], hi)

    m2 = jnp.max(fin, axis=-1, keepdims=True)
    ex2 = jnp.exp(fin - m2)
    out_ref[...] = ex2 / jnp.sum(ex2, axis=-1, keepdims=True)


@functools.partial(jax.jit, static_argnames=())
def kernel(x, Wg1, bg1, Wg2, bg2, W1, b1, W2, b2, W3, b3):
    n, d = x.shape
    g = Wg1.shape[1]
    e = Wg2.shape[1]
    h = W1.shape[2]
    h2 = W2.shape[2]
    c = W3.shape[2]
    ec = e * c
    eg = 4

    f32 = jnp.float32
    Wcat, bcat = pl.pallas_call(
        _collapse_body,
        grid=(e // eg,),
        in_specs=[
            pl.BlockSpec((eg, d, h), lambda i: (i, 0, 0)),
            pl.BlockSpec((eg, h, h2), lambda i: (i, 0, 0)),
            pl.BlockSpec((eg, h2, c), lambda i: (i, 0, 0)),
            pl.BlockSpec((eg, 1, h), lambda i: (i, 0, 0)),
            pl.BlockSpec((eg, 1, h2), lambda i: (i, 0, 0)),
            pl.BlockSpec((eg, 1, c), lambda i: (i, 0, 0)),
        ],
        out_specs=[
            pl.BlockSpec((d, eg * c), lambda i: (0, i)),
            pl.BlockSpec((1, eg * c), lambda i: (0, i)),
        ],
        out_shape=[
            jax.ShapeDtypeStruct((d, ec), f32),
            jax.ShapeDtypeStruct((1, ec), f32),
        ],
    )(W1, W2, W3, b1.reshape(e, 1, h), b2.reshape(e, 1, h2),
      b3.reshape(e, 1, c))
    bcat_ec = bcat.reshape(e, c)

    noise = (_EPS * jax.random.normal(jax.random.key(42), (1,), f32)
             ).reshape(1, 1)
    R = jnp.asarray(
        (np.arange(ec)[None, :] // c) == np.arange(e)[:, None], f32)
    S = jnp.asarray(
        (np.arange(ec)[:, None] % c) == np.arange(c)[None, :], f32)

    out = pl.pallas_call(
        _moe_body,
        grid=(n // _TN,),
        in_specs=[
            pl.BlockSpec((_TN, d), lambda i: (i, 0)),
            pl.BlockSpec((d, g), lambda i: (0, 0)),
            pl.BlockSpec((1, g), lambda i: (0, 0)),
            pl.BlockSpec((g, e), lambda i: (0, 0)),
            pl.BlockSpec((1, e), lambda i: (0, 0)),
            pl.BlockSpec((d, ec), lambda i: (0, 0)),
            pl.BlockSpec((e, c), lambda i: (0, 0)),
            pl.BlockSpec((1, 1), lambda i: (0, 0)),
            pl.BlockSpec((e, ec), lambda i: (0, 0)),
            pl.BlockSpec((ec, c), lambda i: (0, 0)),
        ],
        out_specs=pl.BlockSpec((_TN, c), lambda i: (i, 0)),
        out_shape=jax.ShapeDtypeStruct((n, c), f32),
    )(x, Wg1, bg1.reshape(1, g), Wg2, bg2.reshape(1, e), Wcat, bcat_ec,
      noise, R, S)
    return out


# speedup vs baseline: 1.9906x; 1.2510x over previous
import functools

import jax
import jax.numpy as jnp
import numpy as np
from jax.experimental import pallas as pl

_EPS = 1e-08
_K = 8
_TN = 512


def _mm(a, b, prec):
    return jax.lax.dot_general(
        a, b, (((a.ndim - 1,), (0,)), ((), ())),
        precision=prec, preferred_element_type=jnp.float32)


def _collapse_body(W1_ref, W2_ref, W3_ref, b1_ref, b2_ref, b3_ref,
                   Wcat_ref, bcat_ref):
    hi = jax.lax.Precision.HIGHEST
    eg = W1_ref.shape[0]
    wcols, bcols = [], []
    for i in range(eg):
        W23 = _mm(W2_ref[i], W3_ref[i], hi)
        wcols.append(_mm(W1_ref[i], W23, hi))
        bc = _mm(b1_ref[i], W23, hi) + _mm(b2_ref[i], W3_ref[i], hi) + b3_ref[i]
        bcols.append(bc)
    Wcat_ref[...] = jnp.concatenate(wcols, axis=1)
    bcat_ref[...] = jnp.concatenate(bcols, axis=1)


def _moe_body(x_ref, Wg1_ref, bg1_ref, Wg2_ref, bg2_ref, Wcat_ref, bcat_ref,
              noise_ref, R_ref, S_ref, out_ref):
    hi = jax.lax.Precision.HIGHEST
    df = jax.lax.Precision.DEFAULT
    x = x_ref[...]

    hg = jnp.maximum(_mm(x, Wg1_ref[...], hi) + bg1_ref[...], 0.0)
    logits = _mm(hg, Wg2_ref[...], hi) + bg2_ref[...]
    m = jnp.max(logits, axis=-1, keepdims=True)
    ex = jnp.exp(logits - m)
    w = ex / jnp.sum(ex, axis=-1, keepdims=True)
    w = w + noise_ref[0, 0]

    e_dim = w.shape[-1]
    iota = jax.lax.broadcasted_iota(jnp.int32, w.shape, 1)
    wk = w
    mask = jnp.zeros(w.shape, jnp.bool_)
    for _ in range(_K):
        mx = jnp.max(wk, axis=-1, keepdims=True)
        cand = jnp.where(wk == mx, iota, e_dim)
        sel = jnp.min(cand, axis=-1, keepdims=True)
        first = iota == sel
        mask = jnp.logical_or(mask, first)
        wk = jnp.where(first, -jnp.inf, wk)
    wsel = jnp.where(mask, w, 0.0)

    bf = jnp.bfloat16
    P = _mm(x.astype(bf), Wcat_ref[...], df)
    wide = _mm(wsel.astype(bf), R_ref[...], df)
    fin = _mm((P * wide).astype(bf), S_ref[...], df)
    fin = fin + _mm(wsel, bcat_ref[...], hi)

    m2 = jnp.max(fin, axis=-1, keepdims=True)
    ex2 = jnp.exp(fin - m2)
    out_ref[...] = ex2 / jnp.sum(ex2, axis=-1, keepdims=True)


@functools.partial(jax.jit, static_argnames=())
def kernel(x, Wg1, bg1, Wg2, bg2, W1, b1, W2, b2, W3, b3):
    n, d = x.shape
    g = Wg1.shape[1]
    e = Wg2.shape[1]
    h = W1.shape[2]
    h2 = W2.shape[2]
    c = W3.shape[2]
    ec = e * c
    eg = 4

    f32 = jnp.float32
    Wcat, bcat = pl.pallas_call(
        _collapse_body,
        grid=(e // eg,),
        in_specs=[
            pl.BlockSpec((eg, d, h), lambda i: (i, 0, 0)),
            pl.BlockSpec((eg, h, h2), lambda i: (i, 0, 0)),
            pl.BlockSpec((eg, h2, c), lambda i: (i, 0, 0)),
            pl.BlockSpec((eg, 1, h), lambda i: (i, 0, 0)),
            pl.BlockSpec((eg, 1, h2), lambda i: (i, 0, 0)),
            pl.BlockSpec((eg, 1, c), lambda i: (i, 0, 0)),
        ],
        out_specs=[
            pl.BlockSpec((d, eg * c), lambda i: (0, i)),
            pl.BlockSpec((1, eg * c), lambda i: (0, i)),
        ],
        out_shape=[
            jax.ShapeDtypeStruct((d, ec), f32),
            jax.ShapeDtypeStruct((1, ec), f32),
        ],
    )(W1, W2, W3, b1.reshape(e, 1, h), b2.reshape(e, 1, h2),
      b3.reshape(e, 1, c))
    bcat_ec = bcat.reshape(e, c)

    noise = (_EPS * jax.random.normal(jax.random.key(42), (1,), f32)
             ).reshape(1, 1)
    R = jnp.asarray(
        (np.arange(ec)[None, :] // c) == np.arange(e)[:, None], jnp.bfloat16)
    S = jnp.asarray(
        (np.arange(ec)[:, None] % c) == np.arange(c)[None, :], jnp.bfloat16)

    out = pl.pallas_call(
        _moe_body,
        grid=(n // _TN,),
        in_specs=[
            pl.BlockSpec((_TN, d), lambda i: (i, 0)),
            pl.BlockSpec((d, g), lambda i: (0, 0)),
            pl.BlockSpec((1, g), lambda i: (0, 0)),
            pl.BlockSpec((g, e), lambda i: (0, 0)),
            pl.BlockSpec((1, e), lambda i: (0, 0)),
            pl.BlockSpec((d, ec), lambda i: (0, 0)),
            pl.BlockSpec((e, c), lambda i: (0, 0)),
            pl.BlockSpec((1, 1), lambda i: (0, 0)),
            pl.BlockSpec((e, ec), lambda i: (0, 0)),
            pl.BlockSpec((ec, c), lambda i: (0, 0)),
        ],
        out_specs=pl.BlockSpec((_TN, c), lambda i: (i, 0)),
        out_shape=jax.ShapeDtypeStruct((n, c), f32),
    )(x, Wg1, bg1.reshape(1, g), Wg2, bg2.reshape(1, e),
      Wcat.astype(jnp.bfloat16), bcat_ec, noise, R, S)
    return out


# speedup vs baseline: 2.0435x; 1.0266x over previous
import functools

import jax
import jax.numpy as jnp
import numpy as np
from jax.experimental import pallas as pl

_EPS = 1e-08
_K = 8
_TN = 1024


def _mm(a, b, prec, out_dtype=jnp.float32):
    return jax.lax.dot_general(
        a, b, (((a.ndim - 1,), (0,)), ((), ())),
        precision=prec, preferred_element_type=out_dtype)


def _collapse_body(W1_ref, W2_ref, W3_ref, b1_ref, b2_ref, b3_ref,
                   Wcat_ref, bcat_ref):
    hi = jax.lax.Precision.HIGHEST
    eg = W1_ref.shape[0]
    wcols, bcols = [], []
    for i in range(eg):
        W23 = _mm(W2_ref[i], W3_ref[i], hi)
        wcols.append(_mm(W1_ref[i], W23, hi))
        bc = _mm(b1_ref[i], W23, hi) + _mm(b2_ref[i], W3_ref[i], hi) + b3_ref[i]
        bcols.append(bc)
    Wcat_ref[...] = jnp.concatenate(wcols, axis=1)
    bcat_ref[...] = jnp.concatenate(bcols, axis=1)


def _moe_body(x_ref, Wg1_ref, bg1_ref, Wg2_ref, bg2_ref, Wcat_ref, bcat_ref,
              noise_ref, R_ref, S_ref, out_ref):
    hi = jax.lax.Precision.HIGHEST
    df = jax.lax.Precision.DEFAULT
    x = x_ref[...]

    hg = jnp.maximum(_mm(x, Wg1_ref[...], hi) + bg1_ref[...], 0.0)
    logits = _mm(hg, Wg2_ref[...], hi) + bg2_ref[...]
    m = jnp.max(logits, axis=-1, keepdims=True)
    ex = jnp.exp(logits - m)
    w = ex / jnp.sum(ex, axis=-1, keepdims=True)
    w = w + noise_ref[0, 0]

    e_dim = w.shape[-1]
    iota = jax.lax.broadcasted_iota(jnp.int32, w.shape, 1)
    wk = w
    mask = jnp.zeros(w.shape, jnp.bool_)
    for _ in range(_K):
        mx = jnp.max(wk, axis=-1, keepdims=True)
        cand = jnp.where(wk == mx, iota, e_dim)
        sel = jnp.min(cand, axis=-1, keepdims=True)
        first = iota == sel
        mask = jnp.logical_or(mask, first)
        wk = jnp.where(first, -jnp.inf, wk)
    wsel = jnp.where(mask, w, 0.0)

    bf = jnp.bfloat16
    P = _mm(x.astype(bf), Wcat_ref[...], df)
    wide = _mm(wsel.astype(bf), R_ref[...], df)
    fin = _mm((P * wide).astype(bf), S_ref[...], df)
    fin = fin + _mm(wsel, bcat_ref[...], hi)

    m2 = jnp.max(fin, axis=-1, keepdims=True)
    ex2 = jnp.exp(fin - m2)
    out_ref[...] = ex2 / jnp.sum(ex2, axis=-1, keepdims=True)


@functools.partial(jax.jit, static_argnames=())
def kernel(x, Wg1, bg1, Wg2, bg2, W1, b1, W2, b2, W3, b3):
    n, d = x.shape
    g = Wg1.shape[1]
    e = Wg2.shape[1]
    h = W1.shape[2]
    h2 = W2.shape[2]
    c = W3.shape[2]
    ec = e * c
    eg = 4

    f32 = jnp.float32
    Wcat, bcat = pl.pallas_call(
        _collapse_body,
        grid=(e // eg,),
        in_specs=[
            pl.BlockSpec((eg, d, h), lambda i: (i, 0, 0)),
            pl.BlockSpec((eg, h, h2), lambda i: (i, 0, 0)),
            pl.BlockSpec((eg, h2, c), lambda i: (i, 0, 0)),
            pl.BlockSpec((eg, 1, h), lambda i: (i, 0, 0)),
            pl.BlockSpec((eg, 1, h2), lambda i: (i, 0, 0)),
            pl.BlockSpec((eg, 1, c), lambda i: (i, 0, 0)),
        ],
        out_specs=[
            pl.BlockSpec((d, eg * c), lambda i: (0, i)),
            pl.BlockSpec((1, eg * c), lambda i: (0, i)),
        ],
        out_shape=[
            jax.ShapeDtypeStruct((d, ec), f32),
            jax.ShapeDtypeStruct((1, ec), f32),
        ],
    )(W1, W2, W3, b1.reshape(e, 1, h), b2.reshape(e, 1, h2),
      b3.reshape(e, 1, c))
    bcat_ec = bcat.reshape(e, c)

    noise = (_EPS * jax.random.normal(jax.random.key(42), (1,), f32)
             ).reshape(1, 1)
    R = jnp.asarray(
        (np.arange(ec)[None, :] // c) == np.arange(e)[:, None], jnp.bfloat16)
    S = jnp.asarray(
        (np.arange(ec)[:, None] % c) == np.arange(c)[None, :], jnp.bfloat16)

    out = pl.pallas_call(
        _moe_body,
        grid=(n // _TN,),
        in_specs=[
            pl.BlockSpec((_TN, d), lambda i: (i, 0)),
            pl.BlockSpec((d, g), lambda i: (0, 0)),
            pl.BlockSpec((1, g), lambda i: (0, 0)),
            pl.BlockSpec((g, e), lambda i: (0, 0)),
            pl.BlockSpec((1, e), lambda i: (0, 0)),
            pl.BlockSpec((d, ec), lambda i: (0, 0)),
            pl.BlockSpec((e, c), lambda i: (0, 0)),
            pl.BlockSpec((1, 1), lambda i: (0, 0)),
            pl.BlockSpec((e, ec), lambda i: (0, 0)),
            pl.BlockSpec((ec, c), lambda i: (0, 0)),
        ],
        out_specs=pl.BlockSpec((_TN, c), lambda i: (i, 0)),
        out_shape=jax.ShapeDtypeStruct((n, c), f32),
    )(x, Wg1, bg1.reshape(1, g), Wg2, bg2.reshape(1, e),
      Wcat.astype(jnp.bfloat16), bcat_ec, noise, R, S)
    return out


# speedup vs baseline: 3.2493x; 1.5901x over previous
import functools

import jax
import jax.numpy as jnp
import numpy as np
from jax.experimental import pallas as pl

_EPS = 1e-08
_K = 8
_TN = 1024

_F32 = jnp.float32
_BF16 = jnp.bfloat16


def _dn(a):
    return (((a.ndim - 1,), (0,)), ((), ()))


def _split(a):
    hi = a.astype(_BF16)
    lo = (a - hi.astype(_F32)).astype(_BF16)
    return hi, lo


def _mm3(a_hi, a_lo, b_hi, b_lo):
    df = jax.lax.Precision.DEFAULT
    p0 = jax.lax.dot_general(a_hi, b_hi, _dn(a_hi), precision=df,
                             preferred_element_type=_F32)
    p1 = jax.lax.dot_general(a_hi, b_lo, _dn(a_hi), precision=df,
                             preferred_element_type=_F32)
    p2 = jax.lax.dot_general(a_lo, b_hi, _dn(a_lo), precision=df,
                             preferred_element_type=_F32)
    return p0 + (p1 + p2)


def _collapse_body(W1_ref, W2_ref, W3_ref, b1_ref, b2_ref, b3_ref,
                   Wcatbf_ref, bcat_ref):
    eg = W1_ref.shape[0]
    wcols, bcols = [], []
    for i in range(eg):
        w2h, w2l = _split(W2_ref[i])
        w3h, w3l = _split(W3_ref[i])
        W23 = _mm3(w2h, w2l, w3h, w3l)
        w1h, w1l = _split(W1_ref[i])
        t23h, t23l = _split(W23)
        wcols.append(_mm3(w1h, w1l, t23h, t23l))
        b1h, b1l = _split(b1_ref[i])
        b2h, b2l = _split(b2_ref[i])
        bc = _mm3(b1h, b1l, t23h, t23l) + _mm3(b2h, b2l, w3h, w3l)
        bcols.append(bc + b3_ref[i])
    Wcatbf_ref[...] = jnp.concatenate(wcols, axis=1).astype(_BF16)
    bcat_ref[...] = jnp.concatenate(bcols, axis=1)


def _moe_body(x_ref, Wg1_ref, bg1_ref, Wg2_ref, bg2_ref, Wcat_ref, bcat_ref,
              noise_ref, R_ref, S_ref, out_ref):
    hi = jax.lax.Precision.HIGHEST
    df = jax.lax.Precision.DEFAULT
    x = x_ref[...]

    x_hi, x_lo = _split(x)
    g_hi, g_lo = _split(Wg1_ref[...])
    hg = jnp.maximum(_mm3(x_hi, x_lo, g_hi, g_lo) + bg1_ref[...], 0.0)
    logits = jax.lax.dot_general(hg, Wg2_ref[...], _dn(hg), precision=hi,
                                 preferred_element_type=_F32)
    logits = logits + bg2_ref[...]
    m = jnp.max(logits, axis=-1, keepdims=True)
    ex = jnp.exp(logits - m)
    w = ex / jnp.sum(ex, axis=-1, keepdims=True)
    w = w + noise_ref[0, 0]

    wk = w
    for _ in range(_K):
        mx = jnp.max(wk, axis=-1, keepdims=True)
        wk = jnp.where(wk == mx, -jnp.inf, wk)
    wsel = jnp.where(jnp.isneginf(wk), w, 0.0)

    P = jax.lax.dot_general(x_hi, Wcat_ref[...], _dn(x_hi), precision=df,
                            preferred_element_type=_F32)
    wb = wsel.astype(_BF16)
    wide = jax.lax.dot_general(wb, R_ref[...], _dn(wb), precision=df,
                               preferred_element_type=_F32)
    pw = (P * wide).astype(_BF16)
    fin = jax.lax.dot_general(pw, S_ref[...], _dn(pw), precision=df,
                              preferred_element_type=_F32)
    fin = fin + jax.lax.dot_general(wb, bcat_ref[...].astype(_BF16),
                                    _dn(wb), precision=df,
                                    preferred_element_type=_F32)

    m2 = jnp.max(fin, axis=-1, keepdims=True)
    ex2 = jnp.exp(fin - m2)
    out_ref[...] = ex2 / jnp.sum(ex2, axis=-1, keepdims=True)


@functools.partial(jax.jit, static_argnames=())
def kernel(x, Wg1, bg1, Wg2, bg2, W1, b1, W2, b2, W3, b3):
    n, d = x.shape
    g = Wg1.shape[1]
    e = Wg2.shape[1]
    h = W1.shape[2]
    h2 = W2.shape[2]
    c = W3.shape[2]
    ec = e * c
    eg = 4

    Wcatbf, bcat = pl.pallas_call(
        _collapse_body,
        grid=(e // eg,),
        in_specs=[
            pl.BlockSpec((eg, d, h), lambda i: (i, 0, 0)),
            pl.BlockSpec((eg, h, h2), lambda i: (i, 0, 0)),
            pl.BlockSpec((eg, h2, c), lambda i: (i, 0, 0)),
            pl.BlockSpec((eg, 1, h), lambda i: (i, 0, 0)),
            pl.BlockSpec((eg, 1, h2), lambda i: (i, 0, 0)),
            pl.BlockSpec((eg, 1, c), lambda i: (i, 0, 0)),
        ],
        out_specs=[
            pl.BlockSpec((d, eg * c), lambda i: (0, i)),
            pl.BlockSpec((1, eg * c), lambda i: (0, i)),
        ],
        out_shape=[
            jax.ShapeDtypeStruct((d, ec), _BF16),
            jax.ShapeDtypeStruct((1, ec), _F32),
        ],
    )(W1, W2, W3, b1.reshape(e, 1, h), b2.reshape(e, 1, h2),
      b3.reshape(e, 1, c))
    bcat_ec = bcat.reshape(e, c)

    noise = (_EPS * jax.random.normal(jax.random.key(42), (1,), _F32)
             ).reshape(1, 1)
    R = jnp.asarray(
        (np.arange(ec)[None, :] // c) == np.arange(e)[:, None], _BF16)
    S = jnp.asarray(
        (np.arange(ec)[:, None] % c) == np.arange(c)[None, :], _BF16)

    out = pl.pallas_call(
        _moe_body,
        grid=(n // _TN,),
        in_specs=[
            pl.BlockSpec((_TN, d), lambda i: (i, 0)),
            pl.BlockSpec((d, g), lambda i: (0, 0)),
            pl.BlockSpec((1, g), lambda i: (0, 0)),
            pl.BlockSpec((g, e), lambda i: (0, 0)),
            pl.BlockSpec((1, e), lambda i: (0, 0)),
            pl.BlockSpec((d, ec), lambda i: (0, 0)),
            pl.BlockSpec((e, c), lambda i: (0, 0)),
            pl.BlockSpec((1, 1), lambda i: (0, 0)),
            pl.BlockSpec((e, ec), lambda i: (0, 0)),
            pl.BlockSpec((ec, c), lambda i: (0, 0)),
        ],
        out_specs=pl.BlockSpec((_TN, c), lambda i: (i, 0)),
        out_shape=jax.ShapeDtypeStruct((n, c), _F32),
    )(x, Wg1, bg1.reshape(1, g), Wg2, bg2.reshape(1, e),
      Wcatbf, bcat_ec, noise, R, S)
    return out
